# Initial kernel scaffold; baseline (speedup 1.0000x reference)
import jax
import jax.numpy as jnp
from jax.experimental import pallas as pl


def kernel(x, table, W, b):
    raise NotImplementedError("write your pallas kernel here")


# speedup vs baseline: 12.4031x; 12.4031x over previous
import functools

import jax
import jax.numpy as jnp
from jax import lax
from jax.experimental import pallas as pl
from jax.experimental.pallas import tpu as pltpu
from jax.experimental.pallas import tpu_sc as plsc

_VOCAB = 1_000_000
_EMBED = 32
_BATCH = 16384
_SEQ = 200

_NC, _NS, _L = 2, 16, 16
_NW = _NC * _NS
_ROWS_PER_W = _BATCH // _NW
_COLS = 128
_NCHUNK = _ROWS_PER_W // _COLS

_TV_BLK = 8192


def _tv_body(t_ref, w_ref, b_ref, o_ref):
    t = t_ref[...]
    w = w_ref[...]
    s = jnp.sum(t * w, axis=1)
    o_ref[...] = (s + b_ref[0]) * (1.0 / _SEQ)


def _compute_tv(table, W, b):
    grid = pl.cdiv(_VOCAB, _TV_BLK)
    return pl.pallas_call(
        _tv_body,
        grid=(grid,),
        in_specs=[
            pl.BlockSpec((_TV_BLK, _EMBED), lambda i: (i, 0)),
            pl.BlockSpec((1, _EMBED), lambda i: (0, 0)),
            pl.BlockSpec(memory_space=pltpu.SMEM),
        ],
        out_specs=pl.BlockSpec((_TV_BLK,), lambda i: (i,)),
        out_shape=jax.ShapeDtypeStruct((_VOCAB,), jnp.float32),
    )(table, W, b)


_NFULL = _SEQ // _L
_TAIL = _SEQ - _NFULL * _L


def _sc_body(tv_hbm, x_hbm, out_hbm, idx_v, vals_v, sums_v, sem):
    wid = lax.axis_index("s") * _NC + lax.axis_index("c")
    row0 = wid * _ROWS_PER_W
    lanes = lax.iota(jnp.int32, _L)
    tail_mask = lanes < _TAIL
    last_lane = lanes == (_L - 1)

    def chunk_body(c, carry):
        rbase = row0 + c * _COLS
        pltpu.sync_copy(x_hbm.at[pl.ds(rbase * _SEQ, _COLS * _SEQ)], idx_v)
        pltpu.async_copy(tv_hbm.at[idx_v],
                         vals_v.at[pl.ds(0, _COLS * _SEQ)], sem).wait()

        def row_body(r, carry2):
            base = r * _SEQ
            acc = vals_v[pl.ds(base, _L)]
            for k in range(1, _NFULL):
                acc = acc + vals_v[pl.ds(base + k * _L, _L)]
            tail = vals_v[pl.ds(base + _NFULL * _L, _L)]
            acc = acc + jnp.where(tail_mask, tail, 0.0)
            csum = plsc.cumsum(acc)
            plsc.store_compressed(
                sums_v.at[pl.ds(c * _COLS + r, _L)], csum, mask=last_lane)
            return carry2

        lax.fori_loop(0, _COLS, row_body, 0)
        return carry

    lax.fori_loop(0, _NCHUNK, chunk_body, 0)
    pltpu.sync_copy(sums_v.at[pl.ds(0, _ROWS_PER_W)],
                    out_hbm.at[pl.ds(row0, _ROWS_PER_W)])


_gather_sum = functools.partial(
    pl.kernel,
    out_type=jax.ShapeDtypeStruct((_BATCH,), jnp.float32),
    mesh=plsc.VectorSubcoreMesh(
        core_axis_name="c", subcore_axis_name="s",
        num_cores=_NC, num_subcores=_NS),
    scratch_types=[
        pltpu.VMEM((_COLS * _SEQ,), jnp.int32),
        pltpu.VMEM((_COLS * _SEQ + _L,), jnp.float32),
        pltpu.VMEM((_ROWS_PER_W + _L,), jnp.float32),
        pltpu.SemaphoreType.DMA,
    ],
    compiler_params=pltpu.CompilerParams(needs_layout_passes=False),
)(_sc_body)


def kernel(x, table, W, b):
    tv = _compute_tv(table, W, b)
    out = _gather_sum(tv, x.reshape(-1))
    return out.reshape(_BATCH, 1)


# speedup vs baseline: 13.0149x; 1.0493x over previous
import functools

import jax
import jax.numpy as jnp
from jax import lax
from jax.experimental import pallas as pl
from jax.experimental.pallas import tpu as pltpu
from jax.experimental.pallas import tpu_sc as plsc

_VOCAB = 1_000_000
_EMBED = 32
_BATCH = 16384
_SEQ = 200

_NC, _NS, _L = 2, 16, 16
_NW = _NC * _NS
_ROWS_PER_W = _BATCH // _NW
_COLS = 128
_NCHUNK = _ROWS_PER_W // _COLS

_PACK = 4
_TVROWS = _VOCAB // _PACK
_TV_BLK = 2048


def _tv_body(t_ref, w_ref, b_ref, o_ref):
    t = t_ref[...]
    w = w_ref[...]
    lane = lax.broadcasted_iota(jnp.int32, (_PACK * _EMBED, _PACK), 0)
    grp = lax.broadcasted_iota(jnp.int32, (_PACK * _EMBED, _PACK), 1)
    m = (lane // _EMBED == grp).astype(jnp.float32)
    g = jax.lax.dot_general(t * w, m, (((1,), (0,)), ((), ())),
                            preferred_element_type=jnp.float32)
    o_ref[...] = g + b_ref[0]


def _compute_tv(table, W, b):
    grid = pl.cdiv(_TVROWS, _TV_BLK)
    wt = jnp.tile(W, (1, _PACK)) * (1.0 / _SEQ)
    return pl.pallas_call(
        _tv_body,
        grid=(grid,),
        in_specs=[
            pl.BlockSpec((_TV_BLK, _PACK * _EMBED), lambda i: (i, 0)),
            pl.BlockSpec((1, _PACK * _EMBED), lambda i: (0, 0)),
            pl.BlockSpec(memory_space=pltpu.SMEM),
        ],
        out_specs=pl.BlockSpec((_TV_BLK, _PACK), lambda i: (i, 0)),
        out_shape=jax.ShapeDtypeStruct((_TVROWS, _PACK), jnp.float32),
    )(table.reshape(_TVROWS, _PACK * _EMBED), wt, b * (1.0 / _SEQ))


_NFULL = _SEQ // _L
_TAIL = _SEQ - _NFULL * _L


def _sc_body(tv_hbm, x_hbm, out_hbm, idx_v, vals_v, sums_v, sem):
    wid = lax.axis_index("s") * _NC + lax.axis_index("c")
    row0 = wid * _ROWS_PER_W
    lanes = lax.iota(jnp.int32, _L)
    tail_mask = lanes < _TAIL
    last_lane = lanes == (_L - 1)

    def chunk_body(c, carry):
        rbase = row0 + c * _COLS
        pltpu.sync_copy(x_hbm.at[pl.ds(rbase * _SEQ, _COLS * _SEQ)], idx_v)
        pltpu.async_copy(tv_hbm.at[idx_v],
                         vals_v.at[pl.ds(0, _COLS * _SEQ)], sem).wait()

        def row_body(r, carry2):
            base = r * _SEQ
            acc = vals_v[pl.ds(base, _L)]
            for k in range(1, _NFULL):
                acc = acc + vals_v[pl.ds(base + k * _L, _L)]
            tail = vals_v[pl.ds(base + _NFULL * _L, _L)]
            acc = acc + jnp.where(tail_mask, tail, 0.0)
            csum = plsc.cumsum(acc)
            plsc.store_compressed(
                sums_v.at[pl.ds(c * _COLS + r, _L)], csum, mask=last_lane)
            return carry2

        lax.fori_loop(0, _COLS, row_body, 0)
        return carry

    lax.fori_loop(0, _NCHUNK, chunk_body, 0)
    pltpu.sync_copy(sums_v.at[pl.ds(0, _ROWS_PER_W)],
                    out_hbm.at[pl.ds(row0, _ROWS_PER_W)])


_gather_sum = functools.partial(
    pl.kernel,
    out_type=jax.ShapeDtypeStruct((_BATCH,), jnp.float32),
    mesh=plsc.VectorSubcoreMesh(
        core_axis_name="c", subcore_axis_name="s",
        num_cores=_NC, num_subcores=_NS),
    scratch_types=[
        pltpu.VMEM((_COLS * _SEQ,), jnp.int32),
        pltpu.VMEM((_COLS * _SEQ + _L,), jnp.float32),
        pltpu.VMEM((_ROWS_PER_W + _L,), jnp.float32),
        pltpu.SemaphoreType.DMA,
    ],
    compiler_params=pltpu.CompilerParams(needs_layout_passes=False),
)(_sc_body)


def kernel(x, table, W, b):
    tv = _compute_tv(table, W, b)
    out = _gather_sum(tv.reshape(-1), x.reshape(-1))
    return out.reshape(_BATCH, 1)


# speedup vs baseline: 48.7367x; 3.7447x over previous
import functools

import jax
import jax.numpy as jnp
from jax import lax
from jax.experimental import pallas as pl
from jax.experimental.pallas import tpu as pltpu
from jax.experimental.pallas import tpu_sc as plsc

_VOCAB = 1_000_000
_EMBED = 32
_BATCH = 16384
_SEQ = 200

_NC, _NS, _L = 2, 16, 16
_NW = _NC * _NS
_ROWS_PER_W = _BATCH // _NW
_COLS = 128
_NCHUNK = _ROWS_PER_W // _COLS

_TV_BLKN = 32768


def _tv_body(t_ref, w_ref, b_ref, o_ref):
    t = t_ref[...]
    w = w_ref[...]
    s = jnp.sum(t * w, axis=0)
    o_ref[...] = s + b_ref[0]


def _compute_tv(table, W, b):
    grid = pl.cdiv(_VOCAB, _TV_BLKN)
    return pl.pallas_call(
        _tv_body,
        grid=(grid,),
        in_specs=[
            pl.BlockSpec((_EMBED, _TV_BLKN), lambda i: (0, i)),
            pl.BlockSpec((_EMBED, 1), lambda i: (0, 0)),
            pl.BlockSpec(memory_space=pltpu.SMEM),
        ],
        out_specs=pl.BlockSpec((_TV_BLKN,), lambda i: (i,)),
        out_shape=jax.ShapeDtypeStruct((_VOCAB,), jnp.float32),
    )(table.T, W.T * (1.0 / _SEQ), b * (1.0 / _SEQ))


_NFULL = _SEQ // _L
_TAIL = _SEQ - _NFULL * _L


def _sc_body(tv_hbm, x_hbm, out_hbm, idx_v, vals_v, sums_v, sem):
    wid = lax.axis_index("s") * _NC + lax.axis_index("c")
    row0 = wid * _ROWS_PER_W
    lanes = lax.iota(jnp.int32, _L)
    tail_mask = lanes < _TAIL
    last_lane = lanes == (_L - 1)

    def chunk_body(c, carry):
        rbase = row0 + c * _COLS
        pltpu.sync_copy(x_hbm.at[pl.ds(rbase * _SEQ, _COLS * _SEQ)], idx_v)
        pltpu.async_copy(tv_hbm.at[idx_v],
                         vals_v.at[pl.ds(0, _COLS * _SEQ)], sem).wait()

        def row_body(r, carry2):
            base = r * _SEQ
            acc = vals_v[pl.ds(base, _L)]
            for k in range(1, _NFULL):
                acc = acc + vals_v[pl.ds(base + k * _L, _L)]
            tail = vals_v[pl.ds(base + _NFULL * _L, _L)]
            acc = acc + jnp.where(tail_mask, tail, 0.0)
            csum = plsc.cumsum(acc)
            plsc.store_compressed(
                sums_v.at[pl.ds(c * _COLS + r, _L)], csum, mask=last_lane)
            return carry2

        lax.fori_loop(0, _COLS, row_body, 0)
        return carry

    lax.fori_loop(0, _NCHUNK, chunk_body, 0)
    pltpu.sync_copy(sums_v.at[pl.ds(0, _ROWS_PER_W)],
                    out_hbm.at[pl.ds(row0, _ROWS_PER_W)])


_gather_sum = functools.partial(
    pl.kernel,
    out_type=jax.ShapeDtypeStruct((_BATCH,), jnp.float32),
    mesh=plsc.VectorSubcoreMesh(
        core_axis_name="c", subcore_axis_name="s",
        num_cores=_NC, num_subcores=_NS),
    scratch_types=[
        pltpu.VMEM((_COLS * _SEQ,), jnp.int32),
        pltpu.VMEM((_COLS * _SEQ + _L,), jnp.float32),
        pltpu.VMEM((_ROWS_PER_W + _L,), jnp.float32),
        pltpu.SemaphoreType.DMA,
    ],
    compiler_params=pltpu.CompilerParams(needs_layout_passes=False),
)(_sc_body)


def kernel(x, table, W, b):
    tv = _compute_tv(table, W, b)
    out = _gather_sum(tv, x.reshape(-1))
    return out.reshape(_BATCH, 1)


# speedup vs baseline: 50.4327x; 1.0348x over previous
import functools

import jax
import jax.numpy as jnp
from jax import lax
from jax.experimental import pallas as pl
from jax.experimental.pallas import tpu as pltpu
from jax.experimental.pallas import tpu_sc as plsc

_VOCAB = 1_000_000
_EMBED = 32
_BATCH = 16384
_SEQ = 200

_NC, _NS, _L = 2, 16, 16
_NW = _NC * _NS
_ROWS_PER_W = _BATCH // _NW
_COLS = 64
_NCHUNK = _ROWS_PER_W // _COLS

_TV_BLKN = 32768


def _tv_body(t_ref, w_ref, b_ref, o_ref):
    t = t_ref[...]
    w = w_ref[...]
    s = jnp.sum(t * w, axis=0)
    o_ref[...] = s + b_ref[0]


def _compute_tv(table, W, b):
    grid = pl.cdiv(_VOCAB, _TV_BLKN)
    return pl.pallas_call(
        _tv_body,
        grid=(grid,),
        in_specs=[
            pl.BlockSpec((_EMBED, _TV_BLKN), lambda i: (0, i)),
            pl.BlockSpec((_EMBED, 1), lambda i: (0, 0)),
            pl.BlockSpec(memory_space=pltpu.SMEM),
        ],
        out_specs=pl.BlockSpec((_TV_BLKN,), lambda i: (i,)),
        out_shape=jax.ShapeDtypeStruct((_VOCAB,), jnp.float32),
    )(table.T, W.T * (1.0 / _SEQ), b * (1.0 / _SEQ))


_NFULL = _SEQ // _L
_TAIL = _SEQ - _NFULL * _L


def _sc_body(tv_hbm, x_hbm, out_hbm,
             idx0, idx1, vals0, vals1, sums_v,
             ssem0, ssem1, gsem0, gsem1):
    wid = lax.axis_index("s") * _NC + lax.axis_index("c")
    row0 = wid * _ROWS_PER_W
    lanes = lax.iota(jnp.int32, _L)
    tail_mask = lanes < _TAIL
    last_lane = lanes == (_L - 1)
    idx = (idx0, idx1)
    vals = (vals0, vals1)
    ssem = (ssem0, ssem1)
    gsem = (gsem0, gsem1)

    def stage(c):
        rbase = row0 + c * _COLS
        return pltpu.async_copy(
            x_hbm.at[pl.ds(rbase * _SEQ, _COLS * _SEQ)], idx[c % 2],
            ssem[c % 2])

    def gather(c):
        return pltpu.async_copy(
            tv_hbm.at[idx[c % 2]],
            vals[c % 2].at[pl.ds(0, _COLS * _SEQ)], gsem[c % 2])

    def reduce(c):
        v = vals[c % 2]

        def row_body(r, carry2):
            base = r * _SEQ
            acc = v[pl.ds(base, _L)]
            for k in range(1, _NFULL):
                acc = acc + v[pl.ds(base + k * _L, _L)]
            tail = v[pl.ds(base + _NFULL * _L, _L)]
            acc = acc + jnp.where(tail_mask, tail, 0.0)
            csum = plsc.cumsum(acc)
            plsc.store_compressed(
                sums_v.at[pl.ds(c * _COLS + r, _L)], csum, mask=last_lane)
            return carry2

        lax.fori_loop(0, _COLS, row_body, 0)

    st = stage(0)
    st.wait()
    g_prev = gather(0)
    st = stage(1)
    for c in range(1, _NCHUNK):
        st.wait()
        g_cur = gather(c)
        g_prev.wait()
        reduce(c - 1)
        if c + 1 < _NCHUNK:
            st = stage(c + 1)
        g_prev = g_cur
    g_prev.wait()
    reduce(_NCHUNK - 1)

    pltpu.sync_copy(sums_v.at[pl.ds(0, _ROWS_PER_W)],
                    out_hbm.at[pl.ds(row0, _ROWS_PER_W)])


_gather_sum = functools.partial(
    pl.kernel,
    out_type=jax.ShapeDtypeStruct((_BATCH,), jnp.float32),
    mesh=plsc.VectorSubcoreMesh(
        core_axis_name="c", subcore_axis_name="s",
        num_cores=_NC, num_subcores=_NS),
    scratch_types=[
        pltpu.VMEM((_COLS * _SEQ,), jnp.int32),
        pltpu.VMEM((_COLS * _SEQ,), jnp.int32),
        pltpu.VMEM((_COLS * _SEQ + _L,), jnp.float32),
        pltpu.VMEM((_COLS * _SEQ + _L,), jnp.float32),
        pltpu.VMEM((_ROWS_PER_W + _L,), jnp.float32),
        pltpu.SemaphoreType.DMA,
        pltpu.SemaphoreType.DMA,
        pltpu.SemaphoreType.DMA,
        pltpu.SemaphoreType.DMA,
    ],
    compiler_params=pltpu.CompilerParams(needs_layout_passes=False),
)(_sc_body)


def kernel(x, table, W, b):
    tv = _compute_tv(table, W, b)
    out = _gather_sum(tv, x.reshape(-1))
    return out.reshape(_BATCH, 1)
